# baseline (speedup 1.0000x reference)
import functools

import jax
import jax.numpy as jnp
from jax import lax
from jax.experimental import pallas as pl
from jax.experimental.pallas import tpu as pltpu
from jax.experimental.pallas import tpu_sc as plsc

_B = 1024
_V = 100000
_THR = 0.05
_NC = 2
_NS = 16
_NW = _NC * _NS
_RPW = _B // _NW
_BLK = 400
_NB = _V // _BLK
_NBP = 256
_SCH = 4
_SPAN = _NBP // _SCH
_SV = _SPAN // 16
_RBV = _BLK // 16
_BIGI = jnp.int32(2**31 - 1)


def _tc_blockmax(inp3):
    def body(x_ref, o_ref):
        m = jnp.max(x_ref[...], axis=1)
        m2 = m.reshape(8, _NB)
        o_ref[...] = jnp.pad(
            m2, ((0, 0), (0, _NBP - _NB)), constant_values=-jnp.inf
        )

    return pl.pallas_call(
        body,
        grid=(_B // 8,),
        in_specs=[pl.BlockSpec((8 * _NB, _BLK), lambda i: (i, 0))],
        out_specs=pl.BlockSpec((8, _NBP), lambda i: (i, 0)),
        out_shape=jax.ShapeDtypeStruct((_B, _NBP), jnp.float32),
    )(inp3)


def _merge_top2(a, b):
    a1v, a1i, a2v, a2i = a
    b1v, b1i, b2v, b2i = b
    gt = (b1v > a1v) | ((b1v == a1v) & (b1i < a1i))
    m1 = jnp.where(gt, b1v, a1v)
    i1 = jnp.where(gt, b1i, a1i)
    uv = jnp.where(gt, a1v, a2v)
    ui = jnp.where(gt, a1i, a2i)
    wv = jnp.where(gt, b2v, b1v)
    wi = jnp.where(gt, b2i, b1i)
    gt2 = (wv > uv) | ((wv == uv) & (wi < ui))
    m2 = jnp.where(gt2, wv, uv)
    i2 = jnp.where(gt2, wi, ui)
    return (m1, i1, m2, i2)


def _sc_counts(bm, inp3, tgt):
    mesh = plsc.VectorSubcoreMesh(core_axis_name="c", subcore_axis_name="s")

    @functools.partial(
        pl.kernel,
        mesh=mesh,
        out_type=jax.ShapeDtypeStruct((_NW, 16), jnp.float32),
        scratch_types=[
            pltpu.VMEM((_NBP,), jnp.float32),
            pltpu.VMEM((_NBP,), jnp.float32),
            pltpu.VMEM((1, _BLK), jnp.float32),
            pltpu.VMEM((1, _BLK), jnp.float32),
            pltpu.VMEM((16,), jnp.int32),
            pltpu.VMEM((_RPW,), jnp.int32),
            pltpu.VMEM((16,), jnp.float32),
            pltpu.SemaphoreType.DMA,
            pltpu.SemaphoreType.DMA,
            pltpu.SemaphoreType.DMA,
            pltpu.SemaphoreType.DMA,
        ],
        compiler_params=pltpu.CompilerParams(
            use_tc_tiling_on_sc=False, needs_layout_passes=False
        ),
    )
    def k(bm_hbm, inp_hbm, tgt_hbm, out_hbm, bm0, bm1_, cb1, cb2, idx_v,
          tgt_v, out_v, semb0, semb1, semg1, semg2):
        wid = lax.axis_index("s") * _NC + lax.axis_index("c")
        base = pl.multiple_of(wid * _RPW, _RPW)
        pltpu.sync_copy(tgt_hbm.at[pl.ds(base, _RPW)], tgt_v)
        lane = lax.iota(jnp.int32, 16)

        def bstart(row, buf, sem):
            pltpu.make_async_copy(bm_hbm.at[row], buf, sem).start()

        def bwait(row, buf, sem):
            pltpu.make_async_copy(bm_hbm.at[row], buf, sem).wait()

        bstart(base, bm0, semb0)
        bstart(base + 1, bm1_, semb1)

        def row_half(r, acc, bmv, bsem):
            row = base + r
            bwait(row, bmv, bsem)
            ninf = jnp.full((16,), -jnp.inf, jnp.float32)
            bigi = jnp.full((16,), _BIGI, jnp.int32)

            chains = []
            for j in range(_SCH):
                m1 = ninf
                i1 = bigi
                m2 = ninf
                i2 = bigi
                for v in range(_SV):
                    off = j * _SPAN + v * 16
                    x = bmv[pl.ds(pl.multiple_of(off, 16), 16)]
                    bid = jnp.int32(row * _NB + off) + lane
                    gt1 = x > m1
                    gt2 = x > m2
                    m2 = jnp.where(gt1, m1, jnp.where(gt2, x, m2))
                    i2 = jnp.where(gt1, i1, jnp.where(gt2, bid, i2))
                    m1 = jnp.where(gt1, x, m1)
                    i1 = jnp.where(gt1, bid, i1)
                chains.append((m1, i1, m2, i2))
            bt = chains[0]
            for j in range(1, _SCH):
                bt = _merge_top2(bt, chains[j])
            bm1v, bb1, bm2v, bb2 = bt

            @pl.when(r < _RPW - 2)
            def _():
                bstart(row + 2, bmv, bsem)

            V1 = jnp.max(bm1v)
            eqv = bm1v == V1
            B1 = jnp.min(jnp.where(eqv, bb1, _BIGI))
            win = eqv & (bb1 == B1)
            cv = jnp.where(win, bm2v, bm1v)
            ci = jnp.where(win, bb2, bb1)
            V2 = jnp.max(cv)
            B2 = jnp.min(jnp.where(cv == V2, ci, _BIGI))

            idx_v[pl.ds(0, 16)] = jnp.where(lane < 8, B1, B2)
            g1 = pltpu.make_async_copy(
                inp_hbm.at[idx_v.at[pl.ds(0, 1)]], cb1, semg1
            )
            g2 = pltpu.make_async_copy(
                inp_hbm.at[idx_v.at[pl.ds(8, 1)]], cb2, semg2
            )
            g1.start()
            g2.start()
            g1.wait()
            g2.wait()

            e1 = (B1 - row * _NB) * _BLK
            e2 = (B2 - row * _NB) * _BLK
            s1 = (ninf, bigi, ninf, bigi)
            s2 = (ninf, bigi, ninf, bigi)

            def step(st, eoff, pos, x):
                m1, i1, m2, i2 = st
                gidx = eoff + pos
                gt1 = x > m1
                gt2 = x > m2
                m2 = jnp.where(gt1, m1, jnp.where(gt2, x, m2))
                i2 = jnp.where(gt1, i1, jnp.where(gt2, gidx, i2))
                m1 = jnp.where(gt1, x, m1)
                i1 = jnp.where(gt1, gidx, i1)
                return (m1, i1, m2, i2)

            for v in range(_RBV):
                pos = v * 16 + lane
                s1 = step(s1, e1, pos, cb1[0, pl.ds(v * 16, 16)])
                s2 = step(s2, e2, pos, cb2[0, pl.ds(v * 16, 16)])

            sm = _merge_top2(s1, s2)
            same = B1 == B2
            m1, i1, m2, i2 = tuple(
                jnp.where(same, a, b) for a, b in zip(s1, sm)
            )

            M1 = jnp.max(m1)
            eq = m1 == M1
            I1 = jnp.min(jnp.where(eq, i1, _BIGI))
            winl = eq & (i1 == I1)
            cvl = jnp.where(winl, m2, m1)
            cil = jnp.where(winl, i2, i1)
            M2 = jnp.max(cvl)
            I2 = jnp.min(jnp.where(cvl == M2, cil, _BIGI))
            tv = tgt_v[pl.ds(pl.multiple_of((r // 16) * 16, 16), 16)]
            lsel = lane == (r % 16)
            hit1 = lsel & (tv == I1)
            hit2 = lsel & (tv == I2) & (M1 - M2 < _THR)
            return (
                acc
                + jnp.where(hit1, jnp.float32(1.0), jnp.float32(0.0))
                + jnp.where(hit2, jnp.float32(1.0), jnp.float32(0.0))
            )

        def row_body(rr, acc):
            acc = row_half(2 * rr, acc, bm0, semb0)
            return row_half(2 * rr + 1, acc, bm1_, semb1)

        acc = lax.fori_loop(
            0, _RPW // 2, row_body, jnp.zeros((16,), jnp.float32)
        )
        out_v[...] = acc
        pltpu.sync_copy(out_v, out_hbm.at[wid])

    return k(bm, inp3, tgt)


def _finish(counts):
    def body(x_ref, o_ref):
        o_ref[0] = jnp.float32(1.0) - jnp.sum(x_ref[...]) * jnp.float32(1.0 / _B)

    return pl.pallas_call(
        body,
        out_shape=jax.ShapeDtypeStruct((1,), jnp.float32),
        out_specs=pl.BlockSpec(memory_space=pltpu.SMEM),
    )(counts)


def kernel(input, target):
    inp3 = input.reshape(_B * _NB, _BLK)
    bm = _tc_blockmax(inp3)
    counts = _sc_counts(bm, inp3, target)
    return _finish(counts)[0]


# speedup vs baseline: 1.1152x; 1.1152x over previous
import functools

import jax
import jax.numpy as jnp
from jax import lax
from jax.experimental import pallas as pl
from jax.experimental.pallas import tpu as pltpu
from jax.experimental.pallas import tpu_sc as plsc

_B = 1024
_V = 100000
_THR = 0.05
_NC = 2
_NS = 16
_NW = _NC * _NS
_RPW = _B // _NW
_BLK = 400
_NB = _V // _BLK
_NBP = 256
_TR = 8
_MCH = 4
_BIGI = jnp.int32(2**31 - 1)


def _tc_blocktop2(inp3):
    def body(x_ref, ov1, oi1, ov2, oi2):
        x = x_ref[...]
        iota = lax.broadcasted_iota(jnp.int32, x.shape, 1)
        m = jnp.max(x, axis=1)
        eq = x == m[:, None]
        big = 2**31 - 1
        i1 = jnp.min(jnp.where(eq, iota, big), axis=1)
        dup = jnp.sum(eq.astype(jnp.int32), axis=1) > 1
        sec = jnp.max(jnp.where(eq, -jnp.inf, x), axis=1)
        v2 = jnp.where(dup, m, sec)
        eq2 = (x == v2[:, None]) & (iota != i1[:, None])
        i2 = jnp.min(jnp.where(eq2, iota, big), axis=1)

        def put(o_ref, vals, pad):
            o_ref[...] = jnp.pad(
                vals.reshape(_TR, _NB), ((0, 0), (0, _NBP - _NB)),
                constant_values=pad,
            )

        put(ov1, m, -jnp.inf)
        put(oi1, i1, 2**31 - 1)
        put(ov2, v2, -jnp.inf)
        put(oi2, i2, 2**31 - 1)

    sd = jax.ShapeDtypeStruct
    return pl.pallas_call(
        body,
        grid=(_B // _TR,),
        in_specs=[pl.BlockSpec((_TR * _NB, _BLK), lambda i: (i, 0))],
        out_specs=[pl.BlockSpec((_TR, _NBP), lambda i: (i, 0))] * 4,
        out_shape=[
            sd((_B, _NBP), jnp.float32),
            sd((_B, _NBP), jnp.int32),
            sd((_B, _NBP), jnp.float32),
            sd((_B, _NBP), jnp.int32),
        ],
    )(inp3)


def _merge_top2(a, b):
    a1v, a1i, a2v, a2i = a
    b1v, b1i, b2v, b2i = b
    gt = (b1v > a1v) | ((b1v == a1v) & (b1i < a1i))
    m1 = jnp.where(gt, b1v, a1v)
    i1 = jnp.where(gt, b1i, a1i)
    uv = jnp.where(gt, a1v, a2v)
    ui = jnp.where(gt, a1i, a2i)
    wv = jnp.where(gt, b2v, b1v)
    wi = jnp.where(gt, b2i, b1i)
    gt2 = (wv > uv) | ((wv == uv) & (wi < ui))
    m2 = jnp.where(gt2, wv, uv)
    i2 = jnp.where(gt2, wi, ui)
    return (m1, i1, m2, i2)


def _sc_counts(bv1, bi1, bv2, bi2, tgt):
    mesh = plsc.VectorSubcoreMesh(core_axis_name="c", subcore_axis_name="s")

    @functools.partial(
        pl.kernel,
        mesh=mesh,
        out_type=jax.ShapeDtypeStruct((_NW, 16), jnp.float32),
        scratch_types=(
            [pltpu.VMEM((_NBP,), jnp.float32) for _ in range(4)]
            + [pltpu.VMEM((_NBP,), jnp.int32) for _ in range(4)]
            + [
                pltpu.VMEM((_RPW,), jnp.int32),
                pltpu.VMEM((16,), jnp.float32),
                pltpu.SemaphoreType.DMA,
                pltpu.SemaphoreType.DMA,
            ]
        ),
        compiler_params=pltpu.CompilerParams(
            use_tc_tiling_on_sc=False, needs_layout_passes=False
        ),
    )
    def k(bv1_hbm, bi1_hbm, bv2_hbm, bi2_hbm, tgt_hbm, out_hbm,
          v1a, v1b, v2a, v2b, i1a, i1b, i2a, i2b, tgt_v, out_v,
          sema, semb):
        wid = lax.axis_index("s") * _NC + lax.axis_index("c")
        base = pl.multiple_of(wid * _RPW, _RPW)
        pltpu.sync_copy(tgt_hbm.at[pl.ds(base, _RPW)], tgt_v)
        lane = lax.iota(jnp.int32, 16)
        grp = (
            (v1a, i1a, v2a, i2a, sema),
            (v1b, i1b, v2b, i2b, semb),
        )

        def bstart(row, g):
            gv1, gi1, gv2, gi2, sem = g
            pltpu.make_async_copy(bv1_hbm.at[row], gv1, sem).start()
            pltpu.make_async_copy(bi1_hbm.at[row], gi1, sem).start()
            pltpu.make_async_copy(bv2_hbm.at[row], gv2, sem).start()
            pltpu.make_async_copy(bi2_hbm.at[row], gi2, sem).start()

        def bwait(row, g):
            gv1, gi1, gv2, gi2, sem = g
            pltpu.make_async_copy(bv1_hbm.at[row], gv1, sem).wait()
            pltpu.make_async_copy(bi1_hbm.at[row], gi1, sem).wait()
            pltpu.make_async_copy(bv2_hbm.at[row], gv2, sem).wait()
            pltpu.make_async_copy(bi2_hbm.at[row], gi2, sem).wait()

        bstart(base, grp[0])
        bstart(base + 1, grp[1])

        def row_half(r, acc, g):
            row = base + r
            bwait(row, g)
            gv1, gi1, gv2, gi2, _ = g
            ninf = jnp.full((16,), -jnp.inf, jnp.float32)
            bigi = jnp.full((16,), _BIGI, jnp.int32)

            chains = []
            for j in range(_MCH):
                st = (ninf, bigi, ninf, bigi)
                for v in range(_NBP // 16 // _MCH):
                    off = (j * (_NBP // 16 // _MCH) + v) * 16
                    xb = off + lane
                    ebase = xb * _BLK
                    blk = (
                        gv1[pl.ds(pl.multiple_of(off, 16), 16)],
                        ebase + gi1[pl.ds(pl.multiple_of(off, 16), 16)],
                        gv2[pl.ds(pl.multiple_of(off, 16), 16)],
                        ebase + gi2[pl.ds(pl.multiple_of(off, 16), 16)],
                    )
                    st = _merge_top2(st, blk)
                chains.append(st)
            st = chains[0]
            for j in range(1, _MCH):
                st = _merge_top2(st, chains[j])
            m1, i1, m2, i2 = st

            @pl.when(r < _RPW - 2)
            def _():
                bstart(row + 2, g)

            M1 = jnp.max(m1)
            eq = m1 == M1
            I1 = jnp.min(jnp.where(eq, i1, _BIGI))
            winl = eq & (i1 == I1)
            cvl = jnp.where(winl, m2, m1)
            cil = jnp.where(winl, i2, i1)
            M2 = jnp.max(cvl)
            I2 = jnp.min(jnp.where(cvl == M2, cil, _BIGI))
            tv = tgt_v[pl.ds(pl.multiple_of((r // 16) * 16, 16), 16)]
            lsel = lane == (r % 16)
            hit1 = lsel & (tv == I1)
            hit2 = lsel & (tv == I2) & (M1 - M2 < _THR)
            return (
                acc
                + jnp.where(hit1, jnp.float32(1.0), jnp.float32(0.0))
                + jnp.where(hit2, jnp.float32(1.0), jnp.float32(0.0))
            )

        def row_body(rr, acc):
            acc = row_half(2 * rr, acc, grp[0])
            return row_half(2 * rr + 1, acc, grp[1])

        acc = lax.fori_loop(
            0, _RPW // 2, row_body, jnp.zeros((16,), jnp.float32)
        )
        out_v[...] = acc
        pltpu.sync_copy(out_v, out_hbm.at[wid])

    return k(bv1, bi1, bv2, bi2, tgt)


def _finish(counts):
    def body(x_ref, o_ref):
        o_ref[0] = jnp.float32(1.0) - jnp.sum(x_ref[...]) * jnp.float32(1.0 / _B)

    return pl.pallas_call(
        body,
        out_shape=jax.ShapeDtypeStruct((1,), jnp.float32),
        out_specs=pl.BlockSpec(memory_space=pltpu.SMEM),
    )(counts)


def kernel(input, target):
    inp3 = input.reshape(_B * _NB, _BLK)
    bv1, bi1, bv2, bi2 = _tc_blocktop2(inp3)
    counts = _sc_counts(bv1, bi1, bv2, bi2, target)
    return _finish(counts)[0]


# speedup vs baseline: 1.2739x; 1.1423x over previous
import functools

import jax
import jax.numpy as jnp
from jax import lax
from jax.experimental import pallas as pl
from jax.experimental.pallas import tpu as pltpu
from jax.experimental.pallas import tpu_sc as plsc

_B = 1024
_V = 100000
_THR = 0.05
_NC = 2
_NS = 16
_NW = _NC * _NS
_RPW = _B // _NW
_CHUNK = 20000
_NCHUNK = _V // _CHUNK
_NCH = 5
_CHSPAN = _CHUNK // _NCH
_BLKV = 25
_BLK = _BLKV * 16
_NBLK = _CHSPAN // _BLK
_RCH = 5
_RSPAN = _BLK // _RCH
_RV = _RSPAN // 16
_BIGI = jnp.int32(2**31 - 1)


def _merge_top2(a, b):
    a1v, a1i, a2v, a2i = a
    b1v, b1i, b2v, b2i = b
    gt = (b1v > a1v) | ((b1v == a1v) & (b1i < a1i))
    m1 = jnp.where(gt, b1v, a1v)
    i1 = jnp.where(gt, b1i, a1i)
    uv = jnp.where(gt, a1v, a2v)
    ui = jnp.where(gt, a1i, a2i)
    wv = jnp.where(gt, b2v, b1v)
    wi = jnp.where(gt, b2i, b1i)
    gt2 = (wv > uv) | ((wv == uv) & (wi < ui))
    m2 = jnp.where(gt2, wv, uv)
    i2 = jnp.where(gt2, wi, ui)
    return (m1, i1, m2, i2)


def _chunk_top2(buf, chunk_off, state, lane):
    ninf = jnp.full((16,), -jnp.inf, jnp.float32)
    bigi = jnp.full((16,), _BIGI, jnp.int32)
    init = tuple((ninf, bigi, ninf, bigi) for _ in range(_NCH))

    def body(blk, st):
        out = []
        boff = blk * _BLK
        for j, (bm1, bb1, bm2, bb2) in enumerate(st):
            base = j * _CHSPAN + boff
            mv = buf[pl.ds(pl.multiple_of(base, 16), 16)]
            for v in range(1, _BLKV):
                x = buf[pl.ds(pl.multiple_of(base + v * 16, 16), 16)]
                mv = jnp.maximum(mv, x)
            bid = jnp.int32(j * _CHSPAN) + boff
            gt1 = mv > bm1
            gt2 = mv > bm2
            bm2n = jnp.where(gt1, bm1, jnp.where(gt2, mv, bm2))
            bb2n = jnp.where(gt1, bb1, jnp.where(gt2, bid, bb2))
            bm1n = jnp.where(gt1, mv, bm1)
            bb1n = jnp.where(gt1, bid, bb1)
            out.append((bm1n, bb1n, bm2n, bb2n))
        return tuple(out)

    st = plsc.parallel_loop(0, _NBLK, carry=init, unroll=1)(body)
    bt = st[0]
    for j in range(1, _NCH):
        bt = _merge_top2(bt, st[j])
    bm1, bb1, bm2, bb2 = bt

    V1 = jnp.max(bm1)
    eqv = bm1 == V1
    B1 = jnp.min(jnp.where(eqv, bb1, _BIGI))
    win = eqv & (bb1 == B1)
    cv = jnp.where(win, bm2, bm1)
    ci = jnp.where(win, bb2, bb1)
    V2 = jnp.max(cv)
    B2 = jnp.min(jnp.where(cv == V2, ci, _BIGI))

    def rescan(bb):
        chains = []
        for k in range(_RCH):
            m1 = ninf
            i1 = bigi
            m2 = ninf
            i2 = bigi
            for v in range(_RV):
                lidx = bb + (k * _RSPAN + v * 16) + lane
                x = plsc.load_gather(buf, [lidx])
                gidx = jnp.int32(chunk_off) + lidx
                gt1 = x > m1
                gt2 = x > m2
                m2 = jnp.where(gt1, m1, jnp.where(gt2, x, m2))
                i2 = jnp.where(gt1, i1, jnp.where(gt2, gidx, i2))
                m1 = jnp.where(gt1, x, m1)
                i1 = jnp.where(gt1, gidx, i1)
            chains.append((m1, i1, m2, i2))
        s = chains[0]
        for k in range(1, _RCH):
            s = _merge_top2(s, chains[k])
        return s

    s1 = rescan(B1)
    s2 = rescan(B2)
    sm = _merge_top2(s1, s2)
    same = B1 == B2
    sc = tuple(jnp.where(same, a, b) for a, b in zip(s1, sm))
    return _merge_top2(state, sc)


def _sc_counts(inp, tgt):
    mesh = plsc.VectorSubcoreMesh(core_axis_name="c", subcore_axis_name="s")

    @functools.partial(
        pl.kernel,
        mesh=mesh,
        out_type=jax.ShapeDtypeStruct((_NW, 16), jnp.float32),
        scratch_types=(
            [pltpu.VMEM((_CHUNK,), jnp.float32) for _ in range(_NCHUNK)]
            + [
                pltpu.VMEM((_RPW,), jnp.int32),
                pltpu.VMEM((16,), jnp.float32),
            ]
            + [pltpu.SemaphoreType.DMA for _ in range(_NCHUNK)]
        ),
        compiler_params=pltpu.CompilerParams(
            use_tc_tiling_on_sc=False, needs_layout_passes=False
        ),
    )
    def k(inp_hbm, tgt_hbm, out_hbm, *scratch):
        bufs = scratch[:_NCHUNK]
        tgt_v = scratch[_NCHUNK]
        out_v = scratch[_NCHUNK + 1]
        sems = scratch[_NCHUNK + 2:]
        wid = lax.axis_index("s") * _NC + lax.axis_index("c")
        base = pl.multiple_of(wid * _RPW, _RPW)
        pltpu.sync_copy(tgt_hbm.at[pl.ds(base, _RPW)], tgt_v)
        lane = lax.iota(jnp.int32, 16)

        def start(row, c, buf, sem):
            pltpu.make_async_copy(
                inp_hbm.at[row, pl.ds(c * _CHUNK, _CHUNK)], buf, sem
            ).start()

        def wait(row, c, buf, sem):
            pltpu.make_async_copy(
                inp_hbm.at[row, pl.ds(c * _CHUNK, _CHUNK)], buf, sem
            ).wait()

        for c in range(_NCHUNK):
            start(base, c, bufs[c], sems[c])

        def row_body(r, acc):
            row = base + r
            ninf = jnp.full((16,), -jnp.inf, jnp.float32)
            bigi = jnp.full((16,), _BIGI, jnp.int32)
            state = (ninf, bigi, ninf, bigi)

            for c in range(_NCHUNK):
                wait(row, c, bufs[c], sems[c])
                state = _chunk_top2(bufs[c], c * _CHUNK, state, lane)

                @pl.when(r < _RPW - 1)
                def _(c=c):
                    start(row + 1, c, bufs[c], sems[c])

            m1, i1, m2, i2 = state
            M1 = jnp.max(m1)
            eq = m1 == M1
            I1 = jnp.min(jnp.where(eq, i1, _BIGI))
            win = eq & (i1 == I1)
            cv = jnp.where(win, m2, m1)
            ci = jnp.where(win, i2, i1)
            M2 = jnp.max(cv)
            I2 = jnp.min(jnp.where(cv == M2, ci, _BIGI))
            tvec = tgt_v[pl.ds(pl.multiple_of((r // 16) * 16, 16), 16)]
            lsel = lane == (r % 16)
            hit1 = lsel & (tvec == I1)
            hit2 = lsel & (tvec == I2) & (M1 - M2 < _THR)
            return (
                acc
                + jnp.where(hit1, jnp.float32(1.0), jnp.float32(0.0))
                + jnp.where(hit2, jnp.float32(1.0), jnp.float32(0.0))
            )

        acc = lax.fori_loop(
            0, _RPW, row_body, jnp.zeros((16,), jnp.float32)
        )
        out_v[...] = acc
        pltpu.sync_copy(out_v, out_hbm.at[wid])

    return k(inp, tgt)


def _finish(counts):
    def body(x_ref, o_ref):
        o_ref[0] = jnp.float32(1.0) - jnp.sum(x_ref[...]) * jnp.float32(1.0 / _B)

    return pl.pallas_call(
        body,
        out_shape=jax.ShapeDtypeStruct((1,), jnp.float32),
        out_specs=pl.BlockSpec(memory_space=pltpu.SMEM),
    )(counts)


def kernel(input, target):
    counts = _sc_counts(input, target)
    return _finish(counts)[0]
